# baseline (speedup 1.0000x reference)
import functools

import jax
import jax.numpy as jnp
from jax.experimental import pallas as pl
from jax.experimental.pallas import tpu as pltpu


def _pe_concat_kernel(positions_ref, chromosomes_ref,
                      x_hbm, pe_ref, chrom_ref, out_ref, sem_x,
                      *, seq_len, max_len, c_x, c_pe, c_ch):
    i = pl.program_id(0)
    pos = jnp.clip(positions_ref[i], 0, max_len - seq_len)
    cp_x = pltpu.make_async_copy(x_hbm.at[i], out_ref.at[0, :, 0:c_x], sem_x)
    cp_x.start()
    base = (pos // 8) * 8
    r = pos - base
    slab = pe_ref[pl.ds(base, seq_len + 8), :]
    shift = jnp.where(r == 0, 0, seq_len + 8 - r)
    rolled = pltpu.roll(slab, shift, 0)
    out_ref[0, :, c_x:c_x + c_pe] = rolled[:seq_len, :]
    c = chromosomes_ref[i]
    tbl = chrom_ref[:, :]
    rows = jax.lax.broadcasted_iota(jnp.int32, tbl.shape, 0)
    row = jnp.sum(jnp.where(rows == c, tbl, 0.0), axis=0, keepdims=True)
    out_ref[0, :, c_x + c_pe:c_x + c_pe + c_ch] = jnp.broadcast_to(
        row, (seq_len, c_ch))
    cp_x.wait()


def kernel(x, pe, chrom_table, positions, chromosomes):
    batch, seq_len, c_x = x.shape
    max_len, c_pe = pe.shape
    c_ch = chrom_table.shape[1]
    c_out = c_x + c_pe + c_ch

    pe_padded = jnp.pad(pe, ((0, 8), (0, 0)))

    grid_spec = pltpu.PrefetchScalarGridSpec(
        num_scalar_prefetch=2,
        grid=(batch,),
        in_specs=[
            pl.BlockSpec(memory_space=pl.ANY),
            pl.BlockSpec((max_len + 8, c_pe), lambda i, *_: (0, 0)),
            pl.BlockSpec(chrom_table.shape, lambda i, *_: (0, 0)),
        ],
        out_specs=pl.BlockSpec((1, seq_len, c_out), lambda i, *_: (i, 0, 0)),
        scratch_shapes=[pltpu.SemaphoreType.DMA],
    )

    fn = pl.pallas_call(
        functools.partial(_pe_concat_kernel, seq_len=seq_len, max_len=max_len,
                          c_x=c_x, c_pe=c_pe, c_ch=c_ch),
        grid_spec=grid_spec,
        out_shape=jax.ShapeDtypeStruct((batch, seq_len, c_out), x.dtype),
    )
    return fn(positions.astype(jnp.int32), chromosomes.astype(jnp.int32),
              x, pe_padded, chrom_table)


# speedup vs baseline: 1.1172x; 1.1172x over previous
import functools

import jax
import jax.numpy as jnp
from jax.experimental import pallas as pl
from jax.experimental.pallas import tpu as pltpu

B_BLK = 4


def _pe_concat_kernel(positions_ref, chromosomes_ref,
                      x_hbm, pe_ref, chrom_ref, out_ref, sem_x,
                      *, seq_len, max_len, c_x, c_pe, c_ch):
    i = pl.program_id(0)
    cp_x = pltpu.make_async_copy(
        x_hbm.at[pl.ds(i * B_BLK, B_BLK)],
        out_ref.at[:, :, 0:c_x], sem_x)
    cp_x.start()
    tbl = chrom_ref[:, :]
    rows = jax.lax.broadcasted_iota(jnp.int32, tbl.shape, 0)
    for k in range(B_BLK):
        pos = jnp.clip(positions_ref[i * B_BLK + k], 0, max_len - seq_len)
        base = (pos // 8) * 8
        r = pos - base
        slab = pe_ref[pl.ds(base, seq_len + 8), :]
        shift = jnp.where(r == 0, 0, seq_len + 8 - r)
        rolled = pltpu.roll(slab, shift, 0)
        out_ref[k, :, c_x:c_x + c_pe] = rolled[:seq_len, :]
        c = chromosomes_ref[i * B_BLK + k]
        row = jnp.sum(jnp.where(rows == c, tbl, 0.0), axis=0, keepdims=True)
        out_ref[k, :, c_x + c_pe:c_x + c_pe + c_ch] = jnp.broadcast_to(
            row, (seq_len, c_ch))
    cp_x.wait()


def kernel(x, pe, chrom_table, positions, chromosomes):
    batch, seq_len, c_x = x.shape
    max_len, c_pe = pe.shape
    c_ch = chrom_table.shape[1]
    c_out = c_x + c_pe + c_ch

    pe_padded = jnp.pad(pe, ((0, 8), (0, 0)))

    grid_spec = pltpu.PrefetchScalarGridSpec(
        num_scalar_prefetch=2,
        grid=(batch // B_BLK,),
        in_specs=[
            pl.BlockSpec(memory_space=pl.ANY),
            pl.BlockSpec((max_len + 8, c_pe), lambda i, *_: (0, 0)),
            pl.BlockSpec(chrom_table.shape, lambda i, *_: (0, 0)),
        ],
        out_specs=pl.BlockSpec((B_BLK, seq_len, c_out), lambda i, *_: (i, 0, 0)),
        scratch_shapes=[pltpu.SemaphoreType.DMA],
    )

    fn = pl.pallas_call(
        functools.partial(_pe_concat_kernel, seq_len=seq_len, max_len=max_len,
                          c_x=c_x, c_pe=c_pe, c_ch=c_ch),
        grid_spec=grid_spec,
        out_shape=jax.ShapeDtypeStruct((batch, seq_len, c_out), x.dtype),
    )
    return fn(positions.astype(jnp.int32), chromosomes.astype(jnp.int32),
              x, pe_padded, chrom_table)


# speedup vs baseline: 1.1274x; 1.0092x over previous
import functools

import jax
import jax.numpy as jnp
from jax.experimental import pallas as pl
from jax.experimental.pallas import tpu as pltpu

B_BLK = 4


def _pe_concat_kernel(positions_ref, chromosomes_ref,
                      x_hbm, pe_ref, chrom_ref, out_ref, sem_x,
                      *, seq_len, max_len, c_x, c_pe, c_ch):
    i = pl.program_id(0)
    cp_x = pltpu.make_async_copy(
        x_hbm.at[pl.ds(i * B_BLK, B_BLK)],
        out_ref.at[:, :, 0:c_x], sem_x)
    cp_x.start()
    tbl = chrom_ref[:, :]
    rows = jax.lax.broadcasted_iota(jnp.int32, tbl.shape, 0)
    for k in range(B_BLK):
        pos = jnp.clip(positions_ref[i * B_BLK + k], 0, max_len - seq_len)
        base = jnp.minimum((pos // 8) * 8, max_len - (seq_len + 8))
        r = pos - base
        slab = pe_ref[pl.ds(base, seq_len + 8), :]
        shift = jnp.where(r == 0, 0, seq_len + 8 - r)
        rolled = pltpu.roll(slab, shift, 0)
        out_ref[k, :, c_x:c_x + c_pe] = rolled[:seq_len, :]
        c = chromosomes_ref[i * B_BLK + k]
        row = jnp.sum(jnp.where(rows == c, tbl, 0.0), axis=0, keepdims=True)
        out_ref[k, :, c_x + c_pe:c_x + c_pe + c_ch] = jnp.broadcast_to(
            row, (seq_len, c_ch))
    cp_x.wait()


def kernel(x, pe, chrom_table, positions, chromosomes):
    batch, seq_len, c_x = x.shape
    max_len, c_pe = pe.shape
    c_ch = chrom_table.shape[1]
    c_out = c_x + c_pe + c_ch

    grid_spec = pltpu.PrefetchScalarGridSpec(
        num_scalar_prefetch=2,
        grid=(batch // B_BLK,),
        in_specs=[
            pl.BlockSpec(memory_space=pl.ANY),
            pl.BlockSpec((max_len, c_pe), lambda i, *_: (0, 0)),
            pl.BlockSpec(chrom_table.shape, lambda i, *_: (0, 0)),
        ],
        out_specs=pl.BlockSpec((B_BLK, seq_len, c_out), lambda i, *_: (i, 0, 0)),
        scratch_shapes=[pltpu.SemaphoreType.DMA],
    )

    fn = pl.pallas_call(
        functools.partial(_pe_concat_kernel, seq_len=seq_len, max_len=max_len,
                          c_x=c_x, c_pe=c_pe, c_ch=c_ch),
        grid_spec=grid_spec,
        out_shape=jax.ShapeDtypeStruct((batch, seq_len, c_out), x.dtype),
    )
    return fn(positions.astype(jnp.int32), chromosomes.astype(jnp.int32),
              x, pe, chrom_table)
